# Initial kernel scaffold; baseline (speedup 1.0000x reference)
import jax
import jax.numpy as jnp
from jax.experimental import pallas as pl


def kernel(x, pe_inter, pe_intra, W_time, b_time, W_sub, b_sub, W_dim, b_dim, Wt1, bt1, Wt2, bt2, Wd1, bd1, Wd2, bd2):
    raise NotImplementedError("write your pallas kernel here")


# speedup vs baseline: 1.1764x; 1.1764x over previous
import jax
import jax.numpy as jnp
from jax import lax
from jax.experimental import pallas as pl

SEG_LEN = 128
D1 = 128
D2 = 256
DIM = 8
NODE_NUM = 64
B = 16
PE1 = 32
PE2 = 4
K1 = 12
K2 = 4
K_HOPS = 2
NEG = -3.0e38
F32 = jnp.float32


def _bf(v):
    return v.astype(jnp.bfloat16).astype(F32)


def _topk_mask_rows(sim, k, n):
    col = lax.broadcasted_iota(jnp.int32, sim.shape, sim.ndim - 1)
    work = sim
    adj = jnp.zeros(sim.shape, F32)
    for _ in range(k):
        vmax = jnp.max(work, axis=-1, keepdims=True)
        ismax = work == vmax
        idx = jnp.min(jnp.where(ismax, col, n), axis=-1, keepdims=True)
        onehot = col == idx
        adj = jnp.where(onehot, 1.0, adj)
        work = jnp.where(onehot, NEG, work)
    return adj


def _time_gcn_kernel(x_ref, pe_ref, wt_ref, bt_ref, pei_ref, out_ref, ss_ref):
    s = x_ref[0, 0]
    pe = pe_ref[...]
    b = bt_ref[0]
    pei = pei_ref[0]

    nrm = jnp.sqrt(jnp.sum(s * s, axis=1, keepdims=True))
    sn = s / (nrm + 1e-8)
    sim = lax.dot_general(sn, sn, (((1,), (1,)), ((), ())),
                          preferred_element_type=F32)
    adj = _topk_mask_rows(sim, K1, NODE_NUM)

    row = lax.broadcasted_iota(jnp.int32, (NODE_NUM, NODE_NUM), 0)
    col = lax.broadcasted_iota(jnp.int32, (NODE_NUM, NODE_NUM), 1)
    eye = (row == col).astype(F32)
    ah = adj + eye
    deg = jnp.sum(ah, axis=1, keepdims=True)
    dinv = jnp.where(deg > 0, lax.rsqrt(jnp.maximum(deg, 1e-12)), 0.0)
    dinv_row = jnp.sum(eye * dinv, axis=0, keepdims=True)
    ahat = ah * dinv * dinv_row

    feat = jnp.concatenate([s, pe], axis=1)
    xw = jnp.dot(feat, wt_ref[0], preferred_element_type=F32)
    out = jnp.dot(ahat, xw, preferred_element_type=F32) + b
    out_ref[0, 0] = out
    ss = (jnp.sum(jnp.sum(out * out, axis=1, keepdims=True),
                  axis=0, keepdims=True)
          + jnp.sum(pei * pei, axis=1, keepdims=True))
    ss_ref[0, 0] = jnp.broadcast_to(ss, (1, 128))


def _hsim_kernel(t_ref, w_ref, ss_ref, h_ref, g_ref):
    k = pl.program_id(0)

    @pl.when(k == 0)
    def _init():
        h_ref[...] = jnp.zeros_like(h_ref)
        g_ref[...] = jnp.zeros_like(g_ref)

    t = t_ref[...]
    nrm = jnp.sqrt(ss_ref[:, :1])
    tn = t / (nrm + 1e-8)
    h_ref[...] += jnp.dot(t, w_ref[...], preferred_element_type=F32)
    for b in range(B):
        tb = tn[DIM * b:DIM * (b + 1), :]
        g_ref[b, :, :] += lax.dot_general(
            tb, tb, (((1,), (1,)), ((), ())), preferred_element_type=F32)


def _head_kernel(h_ref, g_ref, ss_ref, pe_ref, wpe_ref, bsub_ref,
                 wdim_ref, bdim_ref, wt1_ref, bt1_ref, wt2_ref, bt2_ref,
                 wd1_ref, bd1_ref, wd2_ref, bd2_ref,
                 task_ref, dom_ref, pool_ref):
    pe = pe_ref[...]
    wpe = wpe_ref[...]

    h_pe = jnp.sum(_bf(pe)[:, :, None] * _bf(wpe)[None, :, :], axis=1)
    h = h_ref[...] + h_pe[None, :, :]

    ss4 = ss_ref[...]
    nrm3 = jnp.sqrt(ss4[:, :, 0, 0:1]) + 1e-8
    pe_n = _bf(pe[None, :, :] / nrm3)
    pe_part = jnp.sum(pe_n[:, :, None, :] * pe_n[:, None, :, :], axis=-1)
    sim2 = g_ref[...] + pe_part
    a2 = _topk_mask_rows(sim2, K2, DIM)

    eye = (lax.broadcasted_iota(jnp.int32, (DIM, DIM), 0)
           == lax.broadcasted_iota(jnp.int32, (DIM, DIM), 1)).astype(F32)

    ab = (a2 > 0).astype(F32)
    m = jnp.broadcast_to(eye[None], (B, DIM, DIM))
    for _ in range(K_HOPS):
        fwd = jnp.sum(m[:, :, :, None] * ab[:, None, :, :], axis=2)
        bwd = jnp.sum(m[:, :, None, :] * ab[:, None, :, :], axis=3)
        m = jnp.clip(m + fwd + bwd, 0.0, 1.0)

    mi = m[:, :, :, None]
    mj = m[:, :, None, :]
    asub = a2[:, None, :, :] * mi * mj + mi * eye[None, None]
    degs = jnp.sum(asub, axis=-1)
    dinvs = jnp.where(degs > 0, lax.rsqrt(jnp.maximum(degs, 1e-12)), 0.0)

    anorm = _bf(asub * dinvs[:, :, :, None] * dinvs[:, :, None, :])
    hb = _bf(h)
    acc = jnp.zeros((B, DIM, DIM, D2), F32)
    for j in range(DIM):
        aj = anorm[:, :, :, j:j + 1]
        hj = hb[:, j:j + 1, :][:, None, :, :]
        acc = acc + aj * hj
    out4 = acc + bsub_ref[...][None, None, None, :]
    masked = jnp.where(m[:, :, :, None] > 0, out4, -1e30)
    sub_out = jnp.max(masked, axis=2)

    ah2 = a2 + eye[None]
    deg2 = jnp.sum(ah2, axis=-1)
    dinv2 = jnp.where(deg2 > 0, lax.rsqrt(jnp.maximum(deg2, 1e-12)), 0.0)
    xw2 = _bf(jnp.dot(sub_out.reshape(B * DIM, D2), wdim_ref[...],
                      preferred_element_type=F32).reshape(B, DIM, D2))
    anorm2 = _bf(ah2 * dinv2[:, :, None] * dinv2[:, None, :])
    acc2 = jnp.zeros((B, DIM, D2), F32)
    for j in range(DIM):
        acc2 = acc2 + anorm2[:, :, j:j + 1] * xw2[:, j:j + 1, :]
    dim_out = acc2 + bdim_ref[...][None, None, :]

    flat = dim_out.reshape(B * DIM, D2)
    hd = jnp.maximum(
        jnp.dot(flat, wd1_ref[...], preferred_element_type=F32)
        + bd1_ref[...][None, :], 0.0)
    dom_ref[...] = jnp.sum(_bf(hd)[:, :, None] * _bf(wd2_ref[...])[None, :, :],
                           axis=1) + bd2_ref[...][None, :]

    pooled = jnp.max(dim_out, axis=1)
    pool_ref[...] = pooled
    ht = jnp.maximum(
        jnp.dot(pooled, wt1_ref[...], preferred_element_type=F32)
        + bt1_ref[...][None, :], 0.0)
    task_ref[...] = jnp.sum(_bf(ht)[:, :, None] * _bf(wt2_ref[...])[None, :, :],
                            axis=1) + bt2_ref[...][None, :]


def kernel(x, pe_inter, pe_intra, W_time, b_time, W_sub, b_sub, W_dim, b_dim,
           Wt1, bt1, Wt2, bt2, Wd1, bd1, Wd2, bd2):
    xs = x.reshape(B, DIM, NODE_NUM, SEG_LEN)

    time_out, sumsq = pl.pallas_call(
        _time_gcn_kernel,
        grid=(B, DIM),
        in_specs=[
            pl.BlockSpec((1, 1, NODE_NUM, SEG_LEN), lambda b, d: (b, d, 0, 0)),
            pl.BlockSpec((NODE_NUM, PE1), lambda b, d: (0, 0)),
            pl.BlockSpec((1, SEG_LEN + PE1, D1), lambda b, d: (d, 0, 0)),
            pl.BlockSpec((1, 1, D1), lambda b, d: (d, 0, 0)),
            pl.BlockSpec((1, 1, PE2), lambda b, d: (d, 0, 0)),
        ],
        out_specs=[
            pl.BlockSpec((1, 1, NODE_NUM, D1), lambda b, d: (b, d, 0, 0)),
            pl.BlockSpec((1, 1, 1, 128), lambda b, d: (b, d, 0, 0)),
        ],
        out_shape=[
            jax.ShapeDtypeStruct((B, DIM, NODE_NUM, D1), F32),
            jax.ShapeDtypeStruct((B, DIM, 1, 128), F32),
        ],
    )(xs, pe_inter, W_time, b_time.reshape(DIM, 1, D1),
      pe_intra.reshape(DIM, 1, PE2))

    nodes2d = time_out.reshape(B * DIM, NODE_NUM * D1)
    ss2d = sumsq.reshape(B * DIM, 128)
    w_main = W_sub[:NODE_NUM * D1, :]
    w_pe = W_sub[NODE_NUM * D1:, :]

    CHUNK = 1024
    nchunks = (NODE_NUM * D1) // CHUNK
    h2d, gram = pl.pallas_call(
        _hsim_kernel,
        grid=(nchunks,),
        in_specs=[
            pl.BlockSpec((B * DIM, CHUNK), lambda k: (0, k)),
            pl.BlockSpec((CHUNK, D2), lambda k: (k, 0)),
            pl.BlockSpec((B * DIM, 128), lambda k: (0, 0)),
        ],
        out_specs=[
            pl.BlockSpec((B * DIM, D2), lambda k: (0, 0)),
            pl.BlockSpec((B, DIM, DIM), lambda k: (0, 0, 0)),
        ],
        out_shape=[
            jax.ShapeDtypeStruct((B * DIM, D2), F32),
            jax.ShapeDtypeStruct((B, DIM, DIM), F32),
        ],
    )(nodes2d, w_main, ss2d)

    h3 = h2d.reshape(B, DIM, D2)

    task, dom, pooled = pl.pallas_call(
        _head_kernel,
        out_shape=[
            jax.ShapeDtypeStruct((B, 2), F32),
            jax.ShapeDtypeStruct((B * DIM, DIM), F32),
            jax.ShapeDtypeStruct((B, D2), F32),
        ],
    )(h3, gram, sumsq, pe_intra, w_pe, b_sub, W_dim, b_dim,
      Wt1, bt1, Wt2, bt2, Wd1, bd1, Wd2, bd2)

    return (task, dom, pooled)


# speedup vs baseline: 5.8408x; 4.9648x over previous
import jax
import jax.numpy as jnp
from jax import lax
from jax.experimental import pallas as pl

SEG_LEN = 128
D1 = 128
D2 = 256
DIM = 8
NODE_NUM = 64
B = 16
PE1 = 32
PE2 = 4
K1 = 12
K2 = 4
K_HOPS = 2
NEG = -3.0e38
F32 = jnp.float32
BN = B * NODE_NUM
BD = B * DIM


def _bf(v):
    return v.astype(jnp.bfloat16).astype(F32)


def _topk_mask_rows(sim, k, n):
    col = lax.broadcasted_iota(jnp.int32, sim.shape, sim.ndim - 1)
    work = sim
    adj = jnp.zeros(sim.shape, F32)
    for _ in range(k):
        vmax = jnp.max(work, axis=-1, keepdims=True)
        ismax = work == vmax
        idx = jnp.min(jnp.where(ismax, col, n), axis=-1, keepdims=True)
        onehot = col == idx
        adj = jnp.where(onehot, 1.0, adj)
        work = jnp.where(onehot, NEG, work)
    return adj


def _time_gcn_kernel(x_ref, pe_ref, wt_ref, bt_ref, pei_ref, out_ref, ss_ref):
    s = x_ref[:, 0].reshape(BN, SEG_LEN)
    pe = pe_ref[...]
    w = wt_ref[0]
    b = bt_ref[0]
    pei = pei_ref[0]

    nrm = jnp.sqrt(jnp.sum(s * s, axis=1, keepdims=True))
    sn = s / (nrm + 1e-8)

    sims = []
    for g in range(B):
        sg = sn[NODE_NUM * g:NODE_NUM * (g + 1), :]
        sims.append(lax.dot_general(sg, sg, (((1,), (1,)), ((), ())),
                                    preferred_element_type=F32))
    sim = jnp.concatenate(sims, axis=0)

    adj = _topk_mask_rows(sim, K1, NODE_NUM)
    row64 = lax.broadcasted_iota(jnp.int32, (BN, NODE_NUM), 0)
    col64 = lax.broadcasted_iota(jnp.int32, (BN, NODE_NUM), 1)
    eye_tall = ((row64 % NODE_NUM) == col64).astype(F32)
    ah = adj + eye_tall
    deg = jnp.sum(ah, axis=1, keepdims=True)
    dinv = jnp.where(deg > 0, lax.rsqrt(jnp.maximum(deg, 1e-12)), 0.0)

    pe_tall = jnp.concatenate([pe] * B, axis=0)
    feat = jnp.concatenate([s, pe_tall], axis=1)
    xw = jnp.dot(feat, w, preferred_element_type=F32)

    eye64 = (lax.broadcasted_iota(jnp.int32, (NODE_NUM, NODE_NUM), 0)
             == lax.broadcasted_iota(jnp.int32, (NODE_NUM, NODE_NUM), 1)
             ).astype(F32)
    for g in range(B):
        sl = slice(NODE_NUM * g, NODE_NUM * (g + 1))
        dinvg = dinv[sl, :]
        dinv_rowg = jnp.sum(eye64 * dinvg, axis=0, keepdims=True)
        ahatg = ah[sl, :] * dinvg * dinv_rowg
        outg = jnp.dot(ahatg, xw[sl, :], preferred_element_type=F32) + b
        out_ref[g, 0] = outg
        ss = (jnp.sum(jnp.sum(outg * outg, axis=1, keepdims=True),
                      axis=0, keepdims=True)
              + jnp.sum(pei * pei, axis=1, keepdims=True))
        ss_ref[g, 0] = jnp.broadcast_to(ss, (1, 128))


def _hsim_kernel(t_ref, w_ref, ss_ref, h_ref, g_ref):
    k = pl.program_id(0)

    @pl.when(k == 0)
    def _init():
        h_ref[...] = jnp.zeros_like(h_ref)
        g_ref[...] = jnp.zeros_like(g_ref)

    t = t_ref[...]
    nrm = jnp.sqrt(ss_ref[:, :1])
    tn = t / (nrm + 1e-8)
    h_ref[...] += jnp.dot(t, w_ref[...], preferred_element_type=F32)
    g_ref[...] += lax.dot_general(tn, tn, (((1,), (1,)), ((), ())),
                                  preferred_element_type=F32)


def _head_kernel(h_ref, g_ref, ss_ref, pe_ref, wpe_ref, bsub_ref,
                 wdim_ref, bdim_ref, wt1_ref, bt1_ref, wt2_ref, bt2_ref,
                 wd1_ref, bd1_ref, wd2_ref, bd2_ref,
                 task_ref, dom_ref, pool_ref):
    pe = pe_ref[...]
    wpe = wpe_ref[...]

    h_pe = jnp.sum(_bf(pe)[:, :, None] * _bf(wpe)[None, :, :], axis=1)
    h = h_ref[...] + h_pe[None, :, :]

    g3 = g_ref[...].reshape(B, DIM, BD)
    lane = lax.broadcasted_iota(jnp.int32, (B, 1, DIM, BD), 3)
    tgt = (DIM * lax.broadcasted_iota(jnp.int32, (B, 1, DIM, BD), 0)
           + lax.broadcasted_iota(jnp.int32, (B, 1, DIM, BD), 2))
    sel = (lane == tgt).astype(F32)
    gblk = jnp.sum(g3[:, :, None, :] * sel, axis=3)

    ss4 = ss_ref[...]
    nrm3 = jnp.sqrt(ss4[:, :, 0, 0:1]) + 1e-8
    pe_n = _bf(pe[None, :, :] / nrm3)
    pe_part = jnp.sum(pe_n[:, :, None, :] * pe_n[:, None, :, :], axis=-1)
    sim2 = gblk + pe_part
    a2 = _topk_mask_rows(sim2, K2, DIM)

    eye = (lax.broadcasted_iota(jnp.int32, (DIM, DIM), 0)
           == lax.broadcasted_iota(jnp.int32, (DIM, DIM), 1)).astype(F32)

    ab = (a2 > 0).astype(F32)
    m = jnp.broadcast_to(eye[None], (B, DIM, DIM))
    for _ in range(K_HOPS):
        fwd = jnp.sum(m[:, :, :, None] * ab[:, None, :, :], axis=2)
        bwd = jnp.sum(m[:, :, None, :] * ab[:, None, :, :], axis=3)
        m = jnp.clip(m + fwd + bwd, 0.0, 1.0)

    mi = m[:, :, :, None]
    mj = m[:, :, None, :]
    asub = a2[:, None, :, :] * mi * mj + mi * eye[None, None]
    degs = jnp.sum(asub, axis=-1)
    dinvs = jnp.where(degs > 0, lax.rsqrt(jnp.maximum(degs, 1e-12)), 0.0)

    anorm = _bf(asub * dinvs[:, :, :, None] * dinvs[:, :, None, :])
    hb = _bf(h)
    acc = jnp.zeros((B, DIM, DIM, D2), F32)
    for j in range(DIM):
        aj = anorm[:, :, :, j:j + 1]
        hj = hb[:, j:j + 1, :][:, None, :, :]
        acc = acc + aj * hj
    out4 = acc + bsub_ref[...][None, None, None, :]
    masked = jnp.where(m[:, :, :, None] > 0, out4, -1e30)
    sub_out = jnp.max(masked, axis=2)

    ah2 = a2 + eye[None]
    deg2 = jnp.sum(ah2, axis=-1)
    dinv2 = jnp.where(deg2 > 0, lax.rsqrt(jnp.maximum(deg2, 1e-12)), 0.0)
    xw2 = _bf(jnp.dot(sub_out.reshape(BD, D2), wdim_ref[...],
                      preferred_element_type=F32).reshape(B, DIM, D2))
    anorm2 = _bf(ah2 * dinv2[:, :, None] * dinv2[:, None, :])
    acc2 = jnp.zeros((B, DIM, D2), F32)
    for j in range(DIM):
        acc2 = acc2 + anorm2[:, :, j:j + 1] * xw2[:, j:j + 1, :]
    dim_out = acc2 + bdim_ref[...][None, None, :]

    flat = dim_out.reshape(BD, D2)
    hd = jnp.maximum(
        jnp.dot(flat, wd1_ref[...], preferred_element_type=F32)
        + bd1_ref[...][None, :], 0.0)
    dom_ref[...] = jnp.sum(_bf(hd)[:, :, None] * _bf(wd2_ref[...])[None, :, :],
                           axis=1) + bd2_ref[...][None, :]

    pooled = jnp.max(dim_out, axis=1)
    pool_ref[...] = pooled
    ht = jnp.maximum(
        jnp.dot(pooled, wt1_ref[...], preferred_element_type=F32)
        + bt1_ref[...][None, :], 0.0)
    task_ref[...] = jnp.sum(_bf(ht)[:, :, None] * _bf(wt2_ref[...])[None, :, :],
                            axis=1) + bt2_ref[...][None, :]


def kernel(x, pe_inter, pe_intra, W_time, b_time, W_sub, b_sub, W_dim, b_dim,
           Wt1, bt1, Wt2, bt2, Wd1, bd1, Wd2, bd2):
    xs = x.reshape(B, DIM, NODE_NUM, SEG_LEN)

    time_out, sumsq = pl.pallas_call(
        _time_gcn_kernel,
        grid=(DIM,),
        in_specs=[
            pl.BlockSpec((B, 1, NODE_NUM, SEG_LEN), lambda d: (0, d, 0, 0)),
            pl.BlockSpec((NODE_NUM, PE1), lambda d: (0, 0)),
            pl.BlockSpec((1, SEG_LEN + PE1, D1), lambda d: (d, 0, 0)),
            pl.BlockSpec((1, 1, D1), lambda d: (d, 0, 0)),
            pl.BlockSpec((1, 1, PE2), lambda d: (d, 0, 0)),
        ],
        out_specs=[
            pl.BlockSpec((B, 1, NODE_NUM, D1), lambda d: (0, d, 0, 0)),
            pl.BlockSpec((B, 1, 1, 128), lambda d: (0, d, 0, 0)),
        ],
        out_shape=[
            jax.ShapeDtypeStruct((B, DIM, NODE_NUM, D1), F32),
            jax.ShapeDtypeStruct((B, DIM, 1, 128), F32),
        ],
    )(xs, pe_inter, W_time, b_time.reshape(DIM, 1, D1),
      pe_intra.reshape(DIM, 1, PE2))

    nodes2d = time_out.reshape(BD, NODE_NUM * D1)
    ss2d = sumsq.reshape(BD, 128)
    w_main = W_sub[:NODE_NUM * D1, :]
    w_pe = W_sub[NODE_NUM * D1:, :]

    CHUNK = 1024
    nchunks = (NODE_NUM * D1) // CHUNK
    h2d, gram = pl.pallas_call(
        _hsim_kernel,
        grid=(nchunks,),
        in_specs=[
            pl.BlockSpec((BD, CHUNK), lambda k: (0, k)),
            pl.BlockSpec((CHUNK, D2), lambda k: (k, 0)),
            pl.BlockSpec((BD, 128), lambda k: (0, 0)),
        ],
        out_specs=[
            pl.BlockSpec((BD, D2), lambda k: (0, 0)),
            pl.BlockSpec((BD, BD), lambda k: (0, 0)),
        ],
        out_shape=[
            jax.ShapeDtypeStruct((BD, D2), F32),
            jax.ShapeDtypeStruct((BD, BD), F32),
        ],
    )(nodes2d, w_main, ss2d)

    h3 = h2d.reshape(B, DIM, D2)

    task, dom, pooled = pl.pallas_call(
        _head_kernel,
        out_shape=[
            jax.ShapeDtypeStruct((B, 2), F32),
            jax.ShapeDtypeStruct((BD, DIM), F32),
            jax.ShapeDtypeStruct((B, D2), F32),
        ],
    )(h3, gram, sumsq, pe_intra, w_pe, b_sub, W_dim, b_dim,
      Wt1, bt1, Wt2, bt2, Wd1, bd1, Wd2, bd2)

    return (task, dom, pooled)
